# baseline (speedup 1.0000x reference)
import functools
import jax
import jax.numpy as jnp
from jax import lax
from jax.experimental import pallas as pl
from jax.experimental.pallas import tpu as pltpu
from jax.experimental.pallas import tpu_sc as plsc

VOC = 1_000_000
D = 64
SCALE = 8.0

NC = 2
NS = 16
NW = NC * NS
B = 16384 * 50
BPW = B // NW
IW = 128
C = 512
NSTREAM = C // IW
NCHUNK = BPW // C
XROWS_PER_CHUNK = C // IW

_mesh = plsc.VectorSubcoreMesh(core_axis_name="c", subcore_axis_name="s")


@functools.partial(
    pl.kernel,
    mesh=_mesh,
    out_type=jax.ShapeDtypeStruct((B, D), jnp.float32),
    compiler_params=pltpu.CompilerParams(use_tc_tiling_on_sc=False),
    scratch_types=[
        pltpu.VMEM((NSTREAM, IW), jnp.int32),
        pltpu.VMEM((C, D), jnp.float32),
        pltpu.SemaphoreType.DMA,
    ],
)
def _emb_lookup(x_hbm, tab_hbm, out_hbm, idx_v, rows_v, gsem):
    wid = lax.axis_index("s") * NC + lax.axis_index("c")
    xrow0 = wid * (BPW // IW)
    obase = wid * BPW

    def chunk(g, carry):
        xr = xrow0 + g * XROWS_PER_CHUNK
        pltpu.sync_copy(x_hbm.at[pl.ds(xr, XROWS_PER_CHUNK)], idx_v)
        cps = []
        for j in range(NSTREAM):
            cps.append(
                pltpu.async_copy(
                    tab_hbm.at[idx_v.at[j]],
                    rows_v.at[pl.ds(j * IW, IW)],
                    gsem,
                )
            )
        for cp in cps:
            cp.wait()

        def srow(i, c2):
            for j in range(D // 16):
                rows_v[i, pl.ds(j * 16, 16)] = (
                    rows_v[i, pl.ds(j * 16, 16)] * SCALE
                )
            return c2

        lax.fori_loop(0, C, srow, 0)
        pltpu.sync_copy(rows_v, out_hbm.at[pl.ds(obase + g * C, C)])
        return carry

    lax.fori_loop(0, NCHUNK, chunk, 0)


def kernel(x, table):
    xf = x.reshape(B // IW, IW).astype(jnp.int32)
    out = _emb_lookup(xf, table)
    return out.reshape(x.shape[0], x.shape[1], D)


# speedup vs baseline: 1.1294x; 1.1294x over previous
import functools
import jax
import jax.numpy as jnp
from jax import lax
from jax.experimental import pallas as pl
from jax.experimental.pallas import tpu as pltpu
from jax.experimental.pallas import tpu_sc as plsc

VOC = 1_000_000
D = 64
SCALE = 8.0

NC = 2
NS = 16
NW = NC * NS
B = 16384 * 50
BPW = B // NW
IW = 128
C = 512
NSTREAM = C // IW
NCHUNK = BPW // C
XROWS = BPW // IW

_mesh = plsc.VectorSubcoreMesh(core_axis_name="c", subcore_axis_name="s")


@functools.partial(
    pl.kernel,
    mesh=_mesh,
    out_type=jax.ShapeDtypeStruct((B, D), jnp.float32),
    compiler_params=pltpu.CompilerParams(use_tc_tiling_on_sc=False),
    scratch_types=[
        pltpu.VMEM((XROWS, IW), jnp.int32),
        pltpu.VMEM((2, C, D), jnp.float32),
        pltpu.SemaphoreType.DMA,
        pltpu.SemaphoreType.DMA,
        pltpu.SemaphoreType.DMA,
        pltpu.SemaphoreType.DMA,
    ],
)
def _emb_lookup(x_hbm, tab_hbm, out_hbm, idx_v, rows_v, g0, g1, s0, s1):
    wid = lax.axis_index("s") * NC + lax.axis_index("c")
    obase = wid * BPW
    gsem = (g0, g1)
    ssem = (s0, s1)

    pltpu.sync_copy(x_hbm.at[pl.ds(wid * XROWS, XROWS)], idx_v)

    def fire_gathers(g, slot):
        return [
            pltpu.async_copy(
                tab_hbm.at[idx_v.at[g * NSTREAM + j]],
                rows_v.at[slot, pl.ds(j * IW, IW)],
                gsem[slot],
            )
            for j in range(NSTREAM)
        ]

    def scale(slot):
        def body(i, c2):
            for r in range(4):
                for j in range(D // 16):
                    rows_v[slot, i * 4 + r, pl.ds(j * 16, 16)] = (
                        rows_v[slot, i * 4 + r, pl.ds(j * 16, 16)] * SCALE
                    )
            return c2

        lax.fori_loop(0, C // 4, body, 0)

    gh = fire_gathers(0, 0)
    sh = [None, None]
    for g in range(NCHUNK):
        slot = g % 2
        if g + 1 < NCHUNK:
            if sh[1 - slot] is not None:
                sh[1 - slot].wait()
            gh_next = fire_gathers(g + 1, 1 - slot)
        else:
            gh_next = None
        for h in gh:
            h.wait()
        scale(slot)
        sh[slot] = pltpu.async_copy(
            rows_v.at[slot], out_hbm.at[pl.ds(obase + g * C, C)], ssem[slot]
        )
        gh = gh_next
    sh[0].wait()
    sh[1].wait()


def kernel(x, table):
    xf = x.reshape(B // IW, IW).astype(jnp.int32)
    out = _emb_lookup(xf, table)
    return out.reshape(x.shape[0], x.shape[1], D)
